# baseline (speedup 1.0000x reference)
import functools

import jax
import jax.numpy as jnp
from jax import lax
from jax.experimental import pallas as pl
from jax.experimental.pallas import tpu as pltpu
from jax.experimental.pallas import tpu_sc as plsc

B = 4
S = 8192
D = 768
L = 16
NC = 2
NS = 16
NW = NC * NS
PER_W = (B * S) // NW
K = 32
NCHUNK = PER_W // K
GROUPS = D // L

_MESH = plsc.VectorSubcoreMesh(
    core_axis_name="c", subcore_axis_name="s", num_cores=NC, num_subcores=NS
)


@functools.partial(
    pl.kernel,
    out_type=jax.ShapeDtypeStruct((B * S, D), jnp.float32),
    mesh=_MESH,
    scratch_types=[
        pltpu.VMEM((NCHUNK, K), jnp.int32),
        pltpu.VMEM((K, D), jnp.float32),
        pltpu.VMEM((K, D), jnp.float32),
        pltpu.VMEM((K, D // 2), jnp.int32),
        pltpu.VMEM((K, D // 2), jnp.int32),
        pltpu.SemaphoreType.DMA,
        pltpu.SemaphoreType.DMA,
        pltpu.SemaphoreType.DMA,
        pltpu.SemaphoreType.DMA,
        pltpu.SemaphoreType.DMA,
        pltpu.SemaphoreType.DMA,
    ],
    compiler_params=pltpu.CompilerParams(needs_layout_passes=False),
)
def _emb_kernel(x_hbm, table_hbm, pe_hbm, out_hbm,
                idx_v, tok0, tok1, pe0, pe1,
                sg0, sg1, sp0, sp1, so0, so1):
    wid = lax.axis_index("s") * NC + lax.axis_index("c")
    base = wid * PER_W
    pos0 = base % S

    toks = (tok0, tok1)
    pes = (pe0, pe1)
    sgs = (sg0, sg1)
    sps = (sp0, sp1)
    sos = (so0, so1)

    pltpu.sync_copy(x_hbm.at[pl.ds(wid * NCHUNK, NCHUNK)], idx_v)

    def start_chunk(j, b):
        pltpu.async_copy(table_hbm.at[idx_v.at[j]], toks[b], sgs[b])
        pltpu.async_copy(pe_hbm.at[pl.ds(pos0 + j * K, K)], pes[b], sps[b])

    start_chunk(0, 0)

    def loop_body(jj, _):
        for b in range(2):
            j = jj * 2 + b
            nb = 1 - b

            @pl.when(j + 1 < NCHUNK)
            def _():
                @pl.when(j >= 1)
                def _():
                    pltpu.make_async_copy(
                        toks[nb], out_hbm.at[pl.ds(base, K)], sos[nb]).wait()
                start_chunk(j + 1, nb)

            pltpu.make_async_copy(
                table_hbm.at[idx_v.at[j]], toks[b], sgs[b]).wait()
            pltpu.make_async_copy(
                pe_hbm.at[pl.ds(pos0, K)], pes[b], sps[b]).wait()

            def row_body(r, _):
                grp = (r // L) * L
                ii = idx_v[j, pl.ds(grp, L)]
                mv = jnp.where(ii != 0, 1.0, 0.0).astype(jnp.float32)
                lane = jnp.full((L, 1), r % L, jnp.int32)
                m = lax.gather(
                    mv, lane,
                    dimension_numbers=lax.GatherDimensionNumbers(
                        offset_dims=(), collapsed_slice_dims=(0,),
                        start_index_map=(0,)),
                    slice_sizes=(1,),
                    mode=lax.GatherScatterMode.PROMISE_IN_BOUNDS)
                for g2 in range(GROUPS // 2):
                    pw = pes[b][r, pl.ds(g2 * L, L)]
                    pb = plsc.bitcast(pw, jnp.bfloat16)
                    plo, phi = plsc.unpack(
                        pb, format=plsc.PackFormat.INTERLEAVED,
                        preferred_element_type=jnp.float32)
                    sl0 = pl.ds(g2 * 2 * L, L)
                    sl1 = pl.ds(g2 * 2 * L + L, L)
                    toks[b][r, sl0] = toks[b][r, sl0] * m + plo
                    toks[b][r, sl1] = toks[b][r, sl1] * m + phi
                return 0

            lax.fori_loop(0, K, row_body, 0, unroll=2)

            pltpu.async_copy(toks[b], out_hbm.at[pl.ds(base + j * K, K)],
                             sos[b])
        return 0

    lax.fori_loop(0, NCHUNK // 2, loop_body, 0)

    pltpu.make_async_copy(tok0, out_hbm.at[pl.ds(base, K)], so0).wait()
    pltpu.make_async_copy(tok1, out_hbm.at[pl.ds(base, K)], so1).wait()


def kernel(x, table, pe):
    x_flat = x.reshape(B * S).astype(jnp.int32).reshape(NW * NCHUNK, K)
    pe_bf = (pe.astype(jnp.bfloat16)
             .reshape(S, D // (2 * L), 2, L)
             .swapaxes(2, 3)
             .reshape(S, D // 2, 2))
    pe_i32 = lax.bitcast_convert_type(pe_bf, jnp.int32)
    out = _emb_kernel(x_flat, table, pe_i32)
    return out.reshape(B, S, D)


# speedup vs baseline: 1.8661x; 1.8661x over previous
import functools

import jax
import jax.numpy as jnp
from jax import lax
from jax.experimental import pallas as pl
from jax.experimental.pallas import tpu as pltpu
from jax.experimental.pallas import tpu_sc as plsc

B = 4
S = 8192
D = 768
L = 16
NC = 2
NS = 16
NW = NC * NS
PER_W = (B * S) // NW
K = 16
NCHUNK = PER_W // K
GROUPS = D // L

_MESH = plsc.VectorSubcoreMesh(
    core_axis_name="c", subcore_axis_name="s", num_cores=NC, num_subcores=NS
)


@functools.partial(
    pl.kernel,
    out_type=jax.ShapeDtypeStruct((B * S, D), jnp.float32),
    mesh=_MESH,
    scratch_types=[
        pltpu.VMEM((NCHUNK, K), jnp.int32),
        pltpu.VMEM((K, D), jnp.float32),
        pltpu.VMEM((K, D), jnp.float32),
        pltpu.VMEM((K, D), jnp.float32),
        pltpu.VMEM((K, D), jnp.float32),
        pltpu.SemaphoreType.DMA,
        pltpu.SemaphoreType.DMA,
        pltpu.SemaphoreType.DMA,
        pltpu.SemaphoreType.DMA,
        pltpu.SemaphoreType.DMA,
        pltpu.SemaphoreType.DMA,
    ],
)
def _emb_kernel(x_hbm, table_hbm, pe_hbm, out_hbm,
                idx_v, tok0, tok1, out0, out1,
                sg0, sg1, sp0, sp1, so0, so1):
    wid = lax.axis_index("s") * NC + lax.axis_index("c")
    base = wid * PER_W
    pos0 = base % S

    toks = (tok0, tok1)
    outs = (out0, out1)
    sgs = (sg0, sg1)
    sps = (sp0, sp1)
    sos = (so0, so1)

    pltpu.sync_copy(x_hbm.at[pl.ds(wid * NCHUNK, NCHUNK)], idx_v)

    def start_chunk(j, b):
        pltpu.async_copy(table_hbm.at[idx_v.at[j]], toks[b], sgs[b])
        pltpu.async_copy(pe_hbm.at[pl.ds(pos0 + j * K, K)], outs[b], sps[b])

    start_chunk(0, 0)

    def loop_body(jj, _):
        for b in range(2):
            j = jj * 2 + b
            nb = 1 - b

            @pl.when(j + 1 < NCHUNK)
            def _():
                @pl.when(j >= 1)
                def _():
                    pltpu.make_async_copy(
                        outs[nb], out_hbm.at[pl.ds(base, K)], sos[nb]).wait()
                start_chunk(j + 1, nb)

            pltpu.make_async_copy(
                table_hbm.at[idx_v.at[j]], toks[b], sgs[b]).wait()
            pltpu.make_async_copy(
                pe_hbm.at[pl.ds(pos0, K)], outs[b], sps[b]).wait()

            def row_body(r, _):
                grp = (r // L) * L
                ii = idx_v[j, pl.ds(grp, L)]
                mv = jnp.where(ii != 0, 1.0, 0.0).astype(jnp.float32)
                lane = jnp.full((L, 1), r % L, jnp.int32)
                m = lax.gather(
                    mv, lane,
                    dimension_numbers=lax.GatherDimensionNumbers(
                        offset_dims=(), collapsed_slice_dims=(0,),
                        start_index_map=(0,)),
                    slice_sizes=(1,),
                    mode=lax.GatherScatterMode.PROMISE_IN_BOUNDS)
                for g in range(GROUPS):
                    sl = pl.ds(g * L, L)
                    plsc.addupdate(outs[b].at[r, sl], toks[b][r, sl] * m)
                return 0

            lax.fori_loop(0, K, row_body, 0, unroll=2)

            pltpu.async_copy(outs[b], out_hbm.at[pl.ds(base + j * K, K)],
                             sos[b])
        return 0

    lax.fori_loop(0, NCHUNK // 2, loop_body, 0)

    pltpu.make_async_copy(out0, out_hbm.at[pl.ds(base, K)], so0).wait()
    pltpu.make_async_copy(out1, out_hbm.at[pl.ds(base, K)], so1).wait()


def kernel(x, table, pe):
    x_flat = x.reshape(B * S).astype(jnp.int32).reshape(NW * NCHUNK, K)
    out = _emb_kernel(x_flat, table, pe)
    return out.reshape(B, S, D)


# speedup vs baseline: 1.9330x; 1.0359x over previous
import functools

import jax
import jax.numpy as jnp
from jax import lax
from jax.experimental import pallas as pl
from jax.experimental.pallas import tpu as pltpu
from jax.experimental.pallas import tpu_sc as plsc

B = 4
S = 8192
D = 768
L = 16
NC = 2
NS = 16
NW = NC * NS
PER_W = (B * S) // NW
K = 32
NCHUNK = PER_W // K
GROUPS = D // L

_MESH = plsc.VectorSubcoreMesh(
    core_axis_name="c", subcore_axis_name="s", num_cores=NC, num_subcores=NS
)


@functools.partial(
    pl.kernel,
    out_type=jax.ShapeDtypeStruct((B * S, D), jnp.float32),
    mesh=_MESH,
    scratch_types=[
        pltpu.VMEM((NCHUNK, K), jnp.int32),
        pltpu.VMEM((K, D), jnp.float32),
        pltpu.VMEM((K, D), jnp.float32),
        pltpu.VMEM((K, D), jnp.float32),
        pltpu.VMEM((K, D), jnp.float32),
        pltpu.SemaphoreType.DMA,
        pltpu.SemaphoreType.DMA,
        pltpu.SemaphoreType.DMA,
        pltpu.SemaphoreType.DMA,
        pltpu.SemaphoreType.DMA,
        pltpu.SemaphoreType.DMA,
    ],
)
def _emb_kernel(x_hbm, table_hbm, pe_hbm, out_hbm,
                idx_v, tok0, tok1, out0, out1,
                sg0, sg1, sp0, sp1, so0, so1):
    wid = lax.axis_index("s") * NC + lax.axis_index("c")
    base = wid * PER_W
    pos0 = base % S

    toks = (tok0, tok1)
    outs = (out0, out1)
    sgs = (sg0, sg1)
    sps = (sp0, sp1)
    sos = (so0, so1)

    pltpu.sync_copy(x_hbm.at[pl.ds(wid * NCHUNK, NCHUNK)], idx_v)

    def start_chunk(j, b):
        pltpu.async_copy(table_hbm.at[idx_v.at[j]], toks[b], sgs[b])
        pltpu.async_copy(pe_hbm.at[pl.ds(pos0 + j * K, K)], outs[b], sps[b])

    start_chunk(0, 0)

    def loop_body(jj, _):
        for b in range(2):
            j = jj * 2 + b
            nb = 1 - b

            @pl.when(j + 1 < NCHUNK)
            def _():
                @pl.when(j >= 1)
                def _():
                    pltpu.make_async_copy(
                        outs[nb], out_hbm.at[pl.ds(base, K)], sos[nb]).wait()
                start_chunk(j + 1, nb)

            pltpu.make_async_copy(
                table_hbm.at[idx_v.at[j]], toks[b], sgs[b]).wait()
            pltpu.make_async_copy(
                pe_hbm.at[pl.ds(pos0, K)], outs[b], sps[b]).wait()

            def row_body(r, _):
                grp = (r // L) * L
                ii = idx_v[j, pl.ds(grp, L)]
                mv = jnp.where(ii != 0, 1.0, 0.0).astype(jnp.float32)
                lane = jnp.full((L, 1), r % L, jnp.int32)
                m = lax.gather(
                    mv, lane,
                    dimension_numbers=lax.GatherDimensionNumbers(
                        offset_dims=(), collapsed_slice_dims=(0,),
                        start_index_map=(0,)),
                    slice_sizes=(1,),
                    mode=lax.GatherScatterMode.PROMISE_IN_BOUNDS)
                for g in range(GROUPS):
                    sl = pl.ds(g * L, L)
                    plsc.addupdate(outs[b].at[r, sl], toks[b][r, sl] * m)
                return 0

            lax.fori_loop(0, K, row_body, 0, unroll=2)

            pltpu.async_copy(outs[b], out_hbm.at[pl.ds(base + j * K, K)],
                             sos[b])
        return 0

    lax.fori_loop(0, NCHUNK // 2, loop_body, 0)

    pltpu.make_async_copy(out0, out_hbm.at[pl.ds(base, K)], so0).wait()
    pltpu.make_async_copy(out1, out_hbm.at[pl.ds(base, K)], so1).wait()


def kernel(x, table, pe):
    x_flat = x.reshape(B * S).astype(jnp.int32).reshape(NW * NCHUNK, K)
    out = _emb_kernel(x_flat, table, pe)
    return out.reshape(B, S, D)
